# baseline (speedup 1.0000x reference)
import jax
import jax.numpy as jnp
from jax.experimental import pallas as pl
from jax.experimental.pallas import tpu as pltpu
from jax.experimental.pallas import tpu_sc as plsc

N = 10000
E = 320000
D = 128
SUB = 1000
G = 64

NC = 2
NS = 16
NW = NC * NS
CHUNK = 128
SUPER = 16
NSUPER = 5
CH = SUPER * NSUPER
E_PAD = NW * CH * CHUNK
N_PAD = 10112
RPT = N_PAD // NS

BN = 1000
NBLK = N // BN


def _segsum_body(h_hbm, src_hbm, dst_hbm, zeros_hbm, out_hbm,
                 is0, is1, idx_d, rows0, rows1, acc, sem0, sem1, semi):
    c = jax.lax.axis_index("c")
    s = jax.lax.axis_index("s")
    wid = c * NS + s
    pltpu.sync_copy(dst_hbm.at[wid], idx_d)
    pltpu.sync_copy(src_hbm.at[wid, 0], is0)
    pltpu.async_copy(h_hbm.at[is0.at[0]], rows0, sem0)
    pltpu.async_copy(h_hbm.at[is0.at[1]], rows1, sem1)
    r0 = s * RPT
    pltpu.sync_copy(zeros_hbm, acc.at[pl.ds(r0, RPT)])
    plsc.subcore_barrier()

    bufs = (is0, is1)
    for t in range(NSUPER):
        cur = bufs[t % 2]
        nxt = bufs[(t + 1) % 2]
        if t + 1 < NSUPER:
            pltpu.async_copy(src_hbm.at[wid, t + 1], nxt, semi)
        base = t * SUPER

        def _step(k, carry, cur=cur, base=base):
            j0 = 2 * k
            j1 = j0 + 1
            pltpu.make_async_copy(h_hbm.at[cur.at[j0]], rows0, sem0).wait()
            pltpu.sync_copy(rows0, acc.at[idx_d.at[base + j0]], add=True)
            pltpu.async_copy(h_hbm.at[cur.at[j0 + 2]], rows0, sem0)
            pltpu.make_async_copy(h_hbm.at[cur.at[j1]], rows1, sem1).wait()
            pltpu.sync_copy(rows1, acc.at[idx_d.at[base + j1]], add=True)
            pltpu.async_copy(h_hbm.at[cur.at[j1 + 2]], rows1, sem1)
            return carry

        jax.lax.fori_loop(0, SUPER // 2 - 1, _step, 0)

        pltpu.make_async_copy(h_hbm.at[cur.at[0]], rows0, sem0).wait()
        pltpu.sync_copy(rows0, acc.at[idx_d.at[base + SUPER - 2]], add=True)
        if t + 1 < NSUPER:
            pltpu.make_async_copy(src_hbm.at[wid, t + 1], nxt, semi).wait()
            pltpu.async_copy(h_hbm.at[nxt.at[0]], rows0, sem0)
        pltpu.make_async_copy(h_hbm.at[cur.at[1]], rows1, sem1).wait()
        pltpu.sync_copy(rows1, acc.at[idx_d.at[base + SUPER - 1]], add=True)
        if t + 1 < NSUPER:
            pltpu.async_copy(h_hbm.at[nxt.at[1]], rows1, sem1)

    plsc.subcore_barrier()
    pltpu.sync_copy(acc.at[pl.ds(r0, RPT)], out_hbm.at[c, pl.ds(r0, RPT)])


_SEGSUM_CACHE = []


def _segsum(h, src_p, dst_p, zeros):
    if not _SEGSUM_CACHE:
        _SEGSUM_CACHE.append(pl.kernel(
            _segsum_body,
            out_type=jax.ShapeDtypeStruct((NC, N_PAD, D), jnp.float32),
            mesh=plsc.VectorSubcoreMesh(
                core_axis_name="c", subcore_axis_name="s"),
            scratch_types=[
                pltpu.VMEM((SUPER, CHUNK), jnp.int32),
                pltpu.VMEM((SUPER, CHUNK), jnp.int32),
                pltpu.VMEM((CH, CHUNK), jnp.int32),
                pltpu.VMEM((CHUNK, D), jnp.float32),
                pltpu.VMEM((CHUNK, D), jnp.float32),
                pltpu.VMEM_SHARED((N_PAD, D), jnp.float32),
                pltpu.SemaphoreType.DMA,
                pltpu.SemaphoreType.DMA,
                pltpu.SemaphoreType.DMA,
            ],
        ))
    return _SEGSUM_CACHE[0](h, src_p, dst_p, zeros)


def _mlp_body(h_ref, agg_ref, w1_ref, b1_ref, w2_ref, b2_ref, o_ref):
    z = h_ref[...] + agg_ref[0] + agg_ref[1]
    y = jnp.maximum(
        jnp.dot(z, w1_ref[...], preferred_element_type=jnp.float32)
        + b1_ref[...], 0.0)
    o_ref[...] = (jnp.dot(y, w2_ref[...], preferred_element_type=jnp.float32)
                  + b2_ref[...])


def _mlp(h, agg, w1, b1, w2, b2):
    return pl.pallas_call(
        _mlp_body,
        grid=(NBLK,),
        in_specs=[
            pl.BlockSpec((BN, D), lambda i: (i, 0)),
            pl.BlockSpec((NC, BN, D), lambda i: (0, i, 0)),
            pl.BlockSpec((D, D), lambda i: (0, 0)),
            pl.BlockSpec((1, D), lambda i: (0, 0)),
            pl.BlockSpec((D, D), lambda i: (0, 0)),
            pl.BlockSpec((1, D), lambda i: (0, 0)),
        ],
        out_specs=pl.BlockSpec((BN, D), lambda i: (i, 0)),
        out_shape=jax.ShapeDtypeStruct((N, D), jnp.float32),
    )(h, agg, w1, b1, w2, b2)


def _pool_body(h_ref, agg_ref, w1_ref, b1_ref, w2_ref, b2_ref,
               n2s_ref, s2g_ref, wh_ref, bh_ref, wr_ref, br_ref,
               wv_ref, bv_ref, out_ref, var_ref, sg_acc):
    i = pl.program_id(0)
    z = h_ref[...] + agg_ref[0] + agg_ref[1]
    y = jnp.maximum(
        jnp.dot(z, w1_ref[...], preferred_element_type=jnp.float32)
        + b1_ref[...], 0.0)
    h3 = (jnp.dot(y, w2_ref[...], preferred_element_type=jnp.float32)
          + b2_ref[...])

    n2s = n2s_ref[0, 0, :]
    oh_ns = (n2s[:, None]
             == jax.lax.broadcasted_iota(jnp.int32, (BN, SUB), 1)
             ).astype(jnp.float32)
    contrib = jax.lax.dot_general(
        oh_ns, h3, (((0,), (0,)), ((), ())),
        preferred_element_type=jnp.float32)

    @pl.when(i == 0)
    def _():
        sg_acc[...] = jnp.zeros_like(sg_acc)

    sg_acc[...] += contrib

    @pl.when(i == pl.num_programs(0) - 1)
    def _():
        s2g = s2g_ref[0, :]
        oh_sg = (s2g[:, None]
                 == jax.lax.broadcasted_iota(jnp.int32, (SUB, G), 1)
                 ).astype(jnp.float32)
        g = jax.lax.dot_general(
            oh_sg, sg_acc[...], (((0,), (0,)), ((), ())),
            preferred_element_type=jnp.float32)
        hid = jnp.maximum(
            jnp.dot(g, wh_ref[...], preferred_element_type=jnp.float32)
            + bh_ref[...], 0.0)
        out_ref[...] = (jnp.dot(hid, wr_ref[...],
                                preferred_element_type=jnp.float32)
                        + br_ref[...])
        var_ref[...] = (jnp.dot(hid, wv_ref[...],
                                preferred_element_type=jnp.float32)
                        + bv_ref[...])


def _pool(h, agg, w1, b1, w2, b2, n2s, s2g, wh, bh, wr, br, wv, bv):
    return pl.pallas_call(
        _pool_body,
        grid=(NBLK,),
        in_specs=[
            pl.BlockSpec((BN, D), lambda i: (i, 0)),
            pl.BlockSpec((NC, BN, D), lambda i: (0, i, 0)),
            pl.BlockSpec((D, D), lambda i: (0, 0)),
            pl.BlockSpec((1, D), lambda i: (0, 0)),
            pl.BlockSpec((D, D), lambda i: (0, 0)),
            pl.BlockSpec((1, D), lambda i: (0, 0)),
            pl.BlockSpec((1, 1, BN), lambda i: (i, 0, 0)),
            pl.BlockSpec((1, SUB), lambda i: (0, 0)),
            pl.BlockSpec((D, D), lambda i: (0, 0)),
            pl.BlockSpec((1, D), lambda i: (0, 0)),
            pl.BlockSpec((D, 1), lambda i: (0, 0)),
            pl.BlockSpec((1, 1), lambda i: (0, 0)),
            pl.BlockSpec((D, 1), lambda i: (0, 0)),
            pl.BlockSpec((1, 1), lambda i: (0, 0)),
        ],
        out_specs=[
            pl.BlockSpec((G, 1), lambda i: (0, 0)),
            pl.BlockSpec((G, 1), lambda i: (0, 0)),
        ],
        out_shape=[
            jax.ShapeDtypeStruct((G, 1), jnp.float32),
            jax.ShapeDtypeStruct((G, 1), jnp.float32),
        ],
        scratch_shapes=[pltpu.VMEM((SUB, D), jnp.float32)],
    )(h, agg, w1, b1, w2, b2, n2s, s2g, wh, bh, wr, br, wv, bv)


def kernel(x, edge_index, node_to_subgraph, subgraph_to_graph,
           W1_0, b1_0, W2_0, b2_0,
           W1_1, b1_1, W2_1, b2_1,
           W1_2, b1_2, W2_2, b2_2,
           Wh, bh, Wr, br, Wv, bv):
    src = edge_index[0]
    dst = edge_index[1]
    pad = E_PAD - E
    src_p = jnp.concatenate(
        [src, jnp.zeros((pad,), jnp.int32)]).reshape(NW, NSUPER, SUPER, CHUNK)
    dst_p = jnp.concatenate(
        [dst, jnp.full((pad,), N, jnp.int32)]).reshape(NW, CH, CHUNK)
    zeros = jnp.zeros((RPT, D), jnp.float32)

    n2s = node_to_subgraph.reshape(NBLK, 1, BN)
    s2g = subgraph_to_graph.reshape(1, SUB)

    h = x
    for (w1, b1, w2, b2) in ((W1_0, b1_0, W2_0, b2_0),
                             (W1_1, b1_1, W2_1, b2_1)):
        agg = _segsum(h, src_p, dst_p, zeros)
        h = _mlp(h, agg, w1, b1.reshape(1, D), w2, b2.reshape(1, D))

    agg = _segsum(h, src_p, dst_p, zeros)
    out, var = _pool(h, agg,
                     W1_2, b1_2.reshape(1, D), W2_2, b2_2.reshape(1, D),
                     n2s, s2g,
                     Wh, bh.reshape(1, D),
                     Wr, br.reshape(1, 1),
                     Wv, bv.reshape(1, 1))
    return (out, var)


# speedup vs baseline: 3.6613x; 3.6613x over previous
import jax
import jax.numpy as jnp
from jax.experimental import pallas as pl
from jax.experimental.pallas import tpu as pltpu
from jax.experimental.pallas import tpu_sc as plsc

N = 10000
E = 320000
D = 128
SUB = 1000
G = 64

NC = 2
NS = 16
NW = NC * NS
CHUNK = 128
SUPER = 16
NSUPER = 5
CH = SUPER * NSUPER
E_PAD = NW * CH * CHUNK
N_PAD = 10112
RPT = N_PAD // NS

BN = 1000
NBLK = N // BN


def _segsum_body(h_hbm, src_hbm, dst_hbm, zeros_hbm, out_hbm,
                 is0, is1, idx_d, rows0, rows1, acc, sem0, sem1, semi):
    c = jax.lax.axis_index("c")
    s = jax.lax.axis_index("s")
    wid = c * NS + s
    pltpu.sync_copy(dst_hbm.at[wid], idx_d)
    pltpu.sync_copy(src_hbm.at[wid, 0], is0)
    pltpu.async_copy(h_hbm.at[is0.at[0]], rows0, sem0)
    pltpu.async_copy(h_hbm.at[is0.at[1]], rows1, sem1)
    r0 = s * RPT
    pltpu.sync_copy(zeros_hbm, acc.at[pl.ds(r0, RPT)])
    plsc.subcore_barrier()

    bufs = (is0, is1)
    for t in range(NSUPER):
        cur = bufs[t % 2]
        nxt = bufs[(t + 1) % 2]
        if t + 1 < NSUPER:
            pltpu.async_copy(src_hbm.at[wid, t + 1], nxt, semi)
        base = t * SUPER

        def _step(k, carry, cur=cur, base=base):
            j0 = 2 * k
            j1 = j0 + 1
            pltpu.make_async_copy(h_hbm.at[cur.at[j0]], rows0, sem0).wait()
            pltpu.sync_copy(rows0, acc.at[idx_d.at[base + j0]], add=True)
            pltpu.async_copy(h_hbm.at[cur.at[j0 + 2]], rows0, sem0)
            pltpu.make_async_copy(h_hbm.at[cur.at[j1]], rows1, sem1).wait()
            pltpu.sync_copy(rows1, acc.at[idx_d.at[base + j1]], add=True)
            pltpu.async_copy(h_hbm.at[cur.at[j1 + 2]], rows1, sem1)
            return carry

        jax.lax.fori_loop(0, SUPER // 2 - 1, _step, 0)

        pltpu.make_async_copy(h_hbm.at[cur.at[0]], rows0, sem0).wait()
        pltpu.sync_copy(rows0, acc.at[idx_d.at[base + SUPER - 2]], add=True)
        if t + 1 < NSUPER:
            pltpu.make_async_copy(src_hbm.at[wid, t + 1], nxt, semi).wait()
            pltpu.async_copy(h_hbm.at[nxt.at[0]], rows0, sem0)
        pltpu.make_async_copy(h_hbm.at[cur.at[1]], rows1, sem1).wait()
        pltpu.sync_copy(rows1, acc.at[idx_d.at[base + SUPER - 1]], add=True)
        if t + 1 < NSUPER:
            pltpu.async_copy(h_hbm.at[nxt.at[1]], rows1, sem1)

    plsc.subcore_barrier()
    pltpu.sync_copy(acc.at[pl.ds(r0, RPT)], out_hbm.at[c, pl.ds(r0, RPT)])


_SEGSUM_CACHE = []


def _segsum(h, src_p, dst_p, zeros):
    if not _SEGSUM_CACHE:
        _SEGSUM_CACHE.append(pl.kernel(
            _segsum_body,
            out_type=jax.ShapeDtypeStruct((NC, N_PAD, D), jnp.float32),
            mesh=plsc.VectorSubcoreMesh(
                core_axis_name="c", subcore_axis_name="s"),
            scratch_types=[
                pltpu.VMEM((SUPER, CHUNK), jnp.int32),
                pltpu.VMEM((SUPER, CHUNK), jnp.int32),
                pltpu.VMEM((CH, CHUNK), jnp.int32),
                pltpu.VMEM((CHUNK, D), jnp.float32),
                pltpu.VMEM((CHUNK, D), jnp.float32),
                pltpu.VMEM_SHARED((N_PAD, D), jnp.float32),
                pltpu.SemaphoreType.DMA,
                pltpu.SemaphoreType.DMA,
                pltpu.SemaphoreType.DMA,
            ],
        ))
    return _SEGSUM_CACHE[0](h, src_p, dst_p, zeros)


def _mlp_body(h_ref, agg_ref, w1_ref, b1_ref, w2_ref, b2_ref, o_ref):
    z = h_ref[...] + agg_ref[0] + agg_ref[1]
    y = jnp.maximum(
        jnp.dot(z, w1_ref[...], preferred_element_type=jnp.float32)
        + b1_ref[...], 0.0)
    o_ref[...] = (jnp.dot(y, w2_ref[...], preferred_element_type=jnp.float32)
                  + b2_ref[...])


def _mlp(h, agg, w1, b1, w2, b2):
    return pl.pallas_call(
        _mlp_body,
        grid=(NBLK,),
        in_specs=[
            pl.BlockSpec((BN, D), lambda i: (i, 0)),
            pl.BlockSpec((NC, BN, D), lambda i: (0, i, 0)),
            pl.BlockSpec((D, D), lambda i: (0, 0)),
            pl.BlockSpec((1, D), lambda i: (0, 0)),
            pl.BlockSpec((D, D), lambda i: (0, 0)),
            pl.BlockSpec((1, D), lambda i: (0, 0)),
        ],
        out_specs=pl.BlockSpec((BN, D), lambda i: (i, 0)),
        out_shape=jax.ShapeDtypeStruct((N, D), jnp.float32),
    )(h, agg, w1, b1, w2, b2)


def _pool_body(h_ref, agg_ref, w1_ref, b1_ref, w2_ref, b2_ref,
               n2s_ref, s2g_ref, wh_ref, bh_ref, wr_ref, br_ref,
               wv_ref, bv_ref, out_ref, var_ref, sg_acc):
    i = pl.program_id(0)
    z = h_ref[...] + agg_ref[0] + agg_ref[1]
    y = jnp.maximum(
        jnp.dot(z, w1_ref[...], preferred_element_type=jnp.float32)
        + b1_ref[...], 0.0)
    h3 = (jnp.dot(y, w2_ref[...], preferred_element_type=jnp.float32)
          + b2_ref[...])

    n2s = n2s_ref[0, 0, :]
    oh_ns = (n2s[:, None]
             == jax.lax.broadcasted_iota(jnp.int32, (BN, SUB), 1)
             ).astype(jnp.float32)
    contrib = jax.lax.dot_general(
        oh_ns, h3, (((0,), (0,)), ((), ())),
        preferred_element_type=jnp.float32)

    @pl.when(i == 0)
    def _():
        sg_acc[...] = jnp.zeros_like(sg_acc)

    sg_acc[...] += contrib

    @pl.when(i == pl.num_programs(0) - 1)
    def _():
        s2g = s2g_ref[0, :]
        oh_sg = (s2g[:, None]
                 == jax.lax.broadcasted_iota(jnp.int32, (SUB, G), 1)
                 ).astype(jnp.float32)
        g = jax.lax.dot_general(
            oh_sg, sg_acc[...], (((0,), (0,)), ((), ())),
            preferred_element_type=jnp.float32)
        hid = jnp.maximum(
            jnp.dot(g, wh_ref[...], preferred_element_type=jnp.float32)
            + bh_ref[...], 0.0)
        out_ref[...] = (jnp.dot(hid, wr_ref[...],
                                preferred_element_type=jnp.float32)
                        + br_ref[...])
        var_ref[...] = (jnp.dot(hid, wv_ref[...],
                                preferred_element_type=jnp.float32)
                        + bv_ref[...])


def _pool(h, agg, w1, b1, w2, b2, n2s, s2g, wh, bh, wr, br, wv, bv):
    return pl.pallas_call(
        _pool_body,
        grid=(NBLK,),
        in_specs=[
            pl.BlockSpec((BN, D), lambda i: (i, 0)),
            pl.BlockSpec((NC, BN, D), lambda i: (0, i, 0)),
            pl.BlockSpec((D, D), lambda i: (0, 0)),
            pl.BlockSpec((1, D), lambda i: (0, 0)),
            pl.BlockSpec((D, D), lambda i: (0, 0)),
            pl.BlockSpec((1, D), lambda i: (0, 0)),
            pl.BlockSpec((1, 1, BN), lambda i: (i, 0, 0)),
            pl.BlockSpec((1, SUB), lambda i: (0, 0)),
            pl.BlockSpec((D, D), lambda i: (0, 0)),
            pl.BlockSpec((1, D), lambda i: (0, 0)),
            pl.BlockSpec((D, 1), lambda i: (0, 0)),
            pl.BlockSpec((1, 1), lambda i: (0, 0)),
            pl.BlockSpec((D, 1), lambda i: (0, 0)),
            pl.BlockSpec((1, 1), lambda i: (0, 0)),
        ],
        out_specs=[
            pl.BlockSpec((G, 1), lambda i: (0, 0)),
            pl.BlockSpec((G, 1), lambda i: (0, 0)),
        ],
        out_shape=[
            jax.ShapeDtypeStruct((G, 1), jnp.float32),
            jax.ShapeDtypeStruct((G, 1), jnp.float32),
        ],
        scratch_shapes=[pltpu.VMEM((SUB, D), jnp.float32)],
    )(h, agg, w1, b1, w2, b2, n2s, s2g, wh, bh, wr, br, wv, bv)


def kernel(x, edge_index, node_to_subgraph, subgraph_to_graph,
           W1_0, b1_0, W2_0, b2_0,
           W1_1, b1_1, W2_1, b2_1,
           W1_2, b1_2, W2_2, b2_2,
           Wh, bh, Wr, br, Wv, bv):
    src = edge_index[0]
    dst = edge_index[1]
    pad = E_PAD - E
    pad_iota = jnp.arange(pad, dtype=jnp.int32)
    pad_src = pad_iota % N
    pad_dst = N + pad_iota % (N_PAD - N)
    src_p = jnp.concatenate(
        [src, pad_src]).reshape(NW, NSUPER, SUPER, CHUNK)
    dst_p = jnp.concatenate(
        [dst, pad_dst]).reshape(NW, CH, CHUNK)
    zeros = jnp.zeros((RPT, D), jnp.float32)

    n2s = node_to_subgraph.reshape(NBLK, 1, BN)
    s2g = subgraph_to_graph.reshape(1, SUB)

    h = x
    for (w1, b1, w2, b2) in ((W1_0, b1_0, W2_0, b2_0),
                             (W1_1, b1_1, W2_1, b2_1)):
        agg = _segsum(h, src_p, dst_p, zeros)
        h = _mlp(h, agg, w1, b1.reshape(1, D), w2, b2.reshape(1, D))

    agg = _segsum(h, src_p, dst_p, zeros)
    out, var = _pool(h, agg,
                     W1_2, b1_2.reshape(1, D), W2_2, b2_2.reshape(1, D),
                     n2s, s2g,
                     Wh, bh.reshape(1, D),
                     Wr, br.reshape(1, 1),
                     Wv, bv.reshape(1, 1))
    return (out, var)
